# baseline (speedup 1.0000x reference)
import functools

import jax
import jax.numpy as jnp
from jax import lax
from jax.experimental import pallas as pl
from jax.experimental.pallas import tpu as pltpu
from jax.experimental.pallas import tpu_sc as plsc

_VOCAB = 1000
_MODEL_DIM = 64
_BATCH = 4096
_HIST = 200

_NC = 2
_NS = 16
_NW = _NC * _NS

_L = 16
_DT = _MODEL_DIM // 8
_BT = _BATCH // 128
_UNITS = _HIST * _DT
_UNITS_PER_W = _UNITS // _NW
_HBT = _BT // 2
_VPAD = 1024


def _body(xt_hbm, tab_hbm, out_hbm, tab_v, xh_v, buf_a, buf_b,
          wsem_a, wsem_b):
  wid = lax.axis_index("s") * _NC + lax.axis_index("c")
  u0 = wid * _UNITS_PER_W

  pltpu.sync_copy(tab_hbm, tab_v)

  def load_xh(h):
    pltpu.sync_copy(xt_hbm.at[h], xh_v)

  def compute_half(dt, s, buf):
    rows = [tab_v.at[dt * 8 + din] for din in range(8)]

    @pl.loop(0, _HBT, unroll=4)
    def _(bt):
      base = (s * _HBT + bt) * 128
      xvs = [xh_v[pl.ds(base + k * _L, _L)] for k in range(8)]
      prev = None
      for k in range(8):
        vals = []
        for din in range(8):
          vals.append(plsc.load_gather(rows[din], [xvs[k]]))
          if prev is not None:
            buf[bt, din, pl.ds((k - 1) * _L, _L)] = prev[din]
        prev = vals
      for din in range(8):
        buf[bt, din, pl.ds(7 * _L, _L)] = prev[din]

  def write_start(h, dt, s, buf, sem):
    pltpu.async_copy(buf, out_hbm.at[h, dt, pl.ds(s * _HBT, _HBT)], sem)

  def write_wait(buf, sem):
    pltpu.make_async_copy(buf, out_hbm.at[0, 0, pl.ds(0, _HBT)], sem).wait()

  h0 = u0 // 8
  dt0 = lax.rem(u0, 8)
  load_xh(h0)
  compute_half(dt0, 0, buf_a)
  write_start(h0, dt0, 0, buf_a, wsem_a)
  compute_half(dt0, 1, buf_b)
  write_start(h0, dt0, 1, buf_b, wsem_b)

  @pl.loop(1, _UNITS_PER_W)
  def _(j):
    u = u0 + j
    h = u // 8
    dt = lax.rem(u, 8)

    @pl.when(dt == 0)
    def _():
      load_xh(h)

    write_wait(buf_a, wsem_a)
    compute_half(dt, 0, buf_a)
    write_start(h, dt, 0, buf_a, wsem_a)
    write_wait(buf_b, wsem_b)
    compute_half(dt, 1, buf_b)
    write_start(h, dt, 1, buf_b, wsem_b)

  write_wait(buf_a, wsem_a)
  write_wait(buf_b, wsem_b)


@jax.jit
def _lookup(xt, tab):
  f = pl.kernel(
      _body,
      out_type=jax.ShapeDtypeStruct((_HIST, _DT, _BT, 8, 128), jnp.float32),
      mesh=plsc.VectorSubcoreMesh(core_axis_name="c", subcore_axis_name="s"),
      scratch_types=[
          pltpu.VMEM((_MODEL_DIM, _VPAD), jnp.float32),
          pltpu.VMEM((_BATCH,), jnp.int32),
          pltpu.VMEM((_HBT, 8, 128), jnp.float32),
          pltpu.VMEM((_HBT, 8, 128), jnp.float32),
          pltpu.SemaphoreType.DMA,
          pltpu.SemaphoreType.DMA,
      ],
      compiler_params=pltpu.CompilerParams(use_tc_tiling_on_sc=False,
                                           needs_layout_passes=False),
  )
  return f(xt, tab)


def kernel(x, text_embed):
  xt = x.T
  tab = jnp.pad(text_embed.T, ((0, 0), (0, _VPAD - _VOCAB)))
  out5 = _lookup(xt, tab)
  return out5.transpose(2, 4, 0, 1, 3).reshape(_BATCH, _HIST, _MODEL_DIM)


# speedup vs baseline: 1.0076x; 1.0076x over previous
import jax
import jax.numpy as jnp
from jax import lax
from jax.experimental import pallas as pl
from jax.experimental.pallas import tpu as pltpu
from jax.experimental.pallas import tpu_sc as plsc

_VOCAB = 1000
_MODEL_DIM = 64
_BATCH = 4096
_HIST = 200

_NC = 2
_NS = 16
_NW = _NC * _NS

_L = 16
_DT = _MODEL_DIM // 8
_BT = _BATCH // 128
_UNITS = _HIST * _DT
_UNITS_PER_W = _UNITS // _NW
_HBT = _BT // 2
_VPAD = 1024


def _body(xt_hbm, tab_hbm, out_hbm, tab_v, xh_v, buf_a, buf_b,
          wsem_a, wsem_b):
  wid = lax.axis_index("s") * _NC + lax.axis_index("c")
  u0 = wid * _UNITS_PER_W

  pltpu.sync_copy(tab_hbm, tab_v)

  def load_xh(h):
    pltpu.sync_copy(xt_hbm.at[h], xh_v)

  def compute_half(dt, s, buf):
    rows = [tab_v.at[dt * 8 + din] for din in range(8)]

    @pl.loop(0, _HBT, unroll=2)
    def _(bt):
      base = (s * _HBT + bt) * 128
      xvs = [xh_v[pl.ds(base + k * _L, _L)] for k in range(8)]
      prev = None
      for k in range(8):
        vals = []
        for din in range(8):
          vals.append(plsc.load_gather(rows[din], [xvs[k]]))
          if prev is not None:
            buf[bt, din, pl.ds((k - 1) * _L, _L)] = prev[din]
        prev = vals
      for din in range(8):
        buf[bt, din, pl.ds(7 * _L, _L)] = prev[din]

  def write_start(h, dt, s, buf, sem):
    pltpu.async_copy(buf, out_hbm.at[h, dt, pl.ds(s * _HBT, _HBT)], sem)

  def write_wait(buf, sem):
    pltpu.make_async_copy(buf, out_hbm.at[0, 0, pl.ds(0, _HBT)], sem).wait()

  h0 = u0 // 8
  dt0 = lax.rem(u0, 8)
  load_xh(h0)
  compute_half(dt0, 0, buf_a)
  write_start(h0, dt0, 0, buf_a, wsem_a)
  compute_half(dt0, 1, buf_b)
  write_start(h0, dt0, 1, buf_b, wsem_b)

  @pl.loop(1, _UNITS_PER_W)
  def _(j):
    u = u0 + j
    h = u // 8
    dt = lax.rem(u, 8)

    @pl.when(dt == 0)
    def _():
      load_xh(h)

    write_wait(buf_a, wsem_a)
    compute_half(dt, 0, buf_a)
    write_start(h, dt, 0, buf_a, wsem_a)
    write_wait(buf_b, wsem_b)
    compute_half(dt, 1, buf_b)
    write_start(h, dt, 1, buf_b, wsem_b)

  write_wait(buf_a, wsem_a)
  write_wait(buf_b, wsem_b)


@jax.jit
def _lookup(xt, tab):
  f = pl.kernel(
      _body,
      out_type=jax.ShapeDtypeStruct((_HIST, _DT, _BT, 8, 128), jnp.float32),
      mesh=plsc.VectorSubcoreMesh(core_axis_name="c", subcore_axis_name="s"),
      scratch_types=[
          pltpu.VMEM((_MODEL_DIM, _VPAD), jnp.float32),
          pltpu.VMEM((_BATCH,), jnp.int32),
          pltpu.VMEM((_HBT, 8, 128), jnp.float32),
          pltpu.VMEM((_HBT, 8, 128), jnp.float32),
          pltpu.SemaphoreType.DMA,
          pltpu.SemaphoreType.DMA,
      ],
      compiler_params=pltpu.CompilerParams(use_tc_tiling_on_sc=False,
                                           needs_layout_passes=False),
  )
  return f(xt, tab)


def kernel(x, text_embed):
  xt = x.T
  tab = jnp.pad(text_embed.T, ((0, 0), (0, _VPAD - _VOCAB)))
  out5 = _lookup(xt, tab)
  return out5.transpose(2, 4, 0, 1, 3).reshape(_BATCH, _HIST, _MODEL_DIM)
